# baseline (speedup 1.0000x reference)
import jax
import jax.numpy as jnp
from jax.experimental import pallas as pl

_BLK_ROWS = 2048


def _mask_kernel(mtop_ref, out_ref):
    mtop = mtop_ref[...]
    cols = jax.lax.broadcasted_iota(jnp.int32, out_ref.shape, 1)
    out_ref[...] = (cols > mtop).astype(jnp.int8)


def kernel(m_top, scores):
    B, H, U, L_K = scores.shape
    rows = B * H * U
    grid = rows // _BLK_ROWS
    mt = m_top.reshape(rows, 1).astype(jnp.int32)
    out = pl.pallas_call(
        _mask_kernel,
        grid=(grid,),
        in_specs=[pl.BlockSpec((_BLK_ROWS, 1), lambda i: (i, 0))],
        out_specs=pl.BlockSpec((_BLK_ROWS, L_K), lambda i: (i, 0)),
        out_shape=jax.ShapeDtypeStruct((rows, L_K), jnp.int8),
    )(mt)
    return out.reshape(B, H, U, L_K).astype(jnp.bool_)


# speedup vs baseline: 1.0127x; 1.0127x over previous
import jax
import jax.numpy as jnp
from jax.experimental import pallas as pl

_BLK_ROWS = 1024


def _mask_kernel(mtop_ref, out_ref):
    mtop = mtop_ref[...]
    cols = jax.lax.broadcasted_iota(jnp.int32, out_ref.shape, 1)
    out_ref[...] = (cols > mtop).astype(jnp.int8)


def kernel(m_top, scores):
    B, H, U, L_K = scores.shape
    rows = B * H * U
    grid = rows // _BLK_ROWS
    mt = m_top.reshape(rows, 1).astype(jnp.int32)
    out = pl.pallas_call(
        _mask_kernel,
        grid=(grid,),
        in_specs=[pl.BlockSpec((_BLK_ROWS, 1), lambda i: (i, 0))],
        out_specs=pl.BlockSpec((_BLK_ROWS, L_K), lambda i: (i, 0)),
        out_shape=jax.ShapeDtypeStruct((rows, L_K), jnp.int8),
    )(mt)
    return out.reshape(B, H, U, L_K).astype(jnp.bool_)
